# Initial kernel scaffold; baseline (speedup 1.0000x reference)
import jax
import jax.numpy as jnp
from jax.experimental import pallas as pl


def kernel(pos, attr, edge_index, W1_lin, W1_src, W1_dst, W1_pos, b1_pos, W2_lin, W2_src, W2_dst, W2_pos, b2_pos):
    raise NotImplementedError("write your pallas kernel here")


# speedup vs baseline: 4.9166x; 4.9166x over previous
import functools

import jax
import jax.numpy as jnp
from jax import lax
from jax.experimental import pallas as pl
from jax.experimental.pallas import tpu as pltpu
from jax.experimental.pallas import tpu_sc as plsc

L = 16
NCHUNK = 4
NCORE = 2
NSUB = 16
NPAD = 51200
EPT = 51200
EB = 128
ZB = 320
ROWS_PT = NPAD // NSUB
BR = 512


def _prep1_body(x_ref, wl_ref, ws_ref, wp_ref, u_ref, q_ref, p_ref):
    x = x_ref[...]
    pos = x[:, 0:3]
    P = jnp.dot(pos, wp_ref[...], preferred_element_type=jnp.float32)
    U = jnp.dot(x, ws_ref[...], preferred_element_type=jnp.float32) + P
    Q = jnp.dot(x, wl_ref[...], preferred_element_type=jnp.float32) - P
    for k in range(NCHUNK):
        sl = slice(L * k, L * k + L)
        u_ref[k] = U[:, sl]
        q_ref[k] = Q[:, sl]
        p_ref[k] = P[:, sl]


def _prep1(xpad, W_lin, W_src, W_pos):
    n = xpad.shape[0]
    cshape = jax.ShapeDtypeStruct((NCHUNK, n, L), jnp.float32)
    full = lambda s: pl.BlockSpec(s, lambda i: (0,) * len(s))
    return pl.pallas_call(
        _prep1_body,
        grid=(n // BR,),
        in_specs=[
            pl.BlockSpec((BR, 6), lambda i: (i, 0)),
            full((6, 64)), full((6, 64)), full((3, 64)),
        ],
        out_specs=[pl.BlockSpec((NCHUNK, BR, L), lambda i: (0, i, 0))] * 3,
        out_shape=[cshape] * 3,
    )(xpad, W_lin, W_src, W_pos)


def _combine2_body(acc_ref, q1_ref, p1_ref, b1_ref, x_ref, wl_ref, ws_ref,
                   wp_ref, u_ref, q_ref, p_ref):
    parts = []
    b1 = b1_ref[...]
    for k in range(NCHUNK):
        den = acc_ref[k][:, 0:L] + 1.0
        num = acc_ref[k][:, L:2 * L] + q1_ref[k]
        parts.append(num / den + p1_ref[k] + b1[:, L * k:L * k + L])
    h = jnp.concatenate(parts, axis=1)
    pos = x_ref[...][:, 0:3]
    P = jnp.dot(pos, wp_ref[...], preferred_element_type=jnp.float32)
    U = jnp.dot(h, ws_ref[...], preferred_element_type=jnp.float32) + P
    Q = jnp.dot(h, wl_ref[...], preferred_element_type=jnp.float32) - P
    for k in range(NCHUNK):
        sl = slice(L * k, L * k + L)
        u_ref[k] = U[:, sl]
        q_ref[k] = Q[:, sl]
        p_ref[k] = P[:, sl]


def _combine2(acc, q1c, p1c, b1, xpad, W_lin, W_src, W_pos):
    n = xpad.shape[0]
    cshape = jax.ShapeDtypeStruct((NCHUNK, n, L), jnp.float32)
    full = lambda s: pl.BlockSpec(s, lambda i: (0,) * len(s))
    cblock = pl.BlockSpec((NCHUNK, BR, L), lambda i: (0, i, 0))
    return pl.pallas_call(
        _combine2_body,
        grid=(n // BR,),
        in_specs=[
            pl.BlockSpec((NCHUNK, BR, 2 * L), lambda i: (0, i, 0)),
            cblock, cblock, full((1, 64)),
            pl.BlockSpec((BR, 6), lambda i: (i, 0)),
            full((64, 64)), full((64, 64)), full((3, 64)),
        ],
        out_specs=[cblock] * 3,
        out_shape=[cshape] * 3,
    )(acc, q1c, p1c, b1, xpad, W_lin, W_src, W_pos)


def _final_body(acc_ref, q2_ref, p2_ref, b2_ref, o_ref):
    b2 = b2_ref[...]
    for k in range(NCHUNK):
        den = acc_ref[k][:, 0:L] + 1.0
        num = acc_ref[k][:, L:2 * L] + q2_ref[k]
        o_ref[:, L * k:L * k + L] = num / den + p2_ref[k] + b2[:, L * k:L * k + L]


def _final(acc, q2c, p2c, b2):
    n = acc.shape[1]
    full = lambda s: pl.BlockSpec(s, lambda i: (0,) * len(s))
    cblock = pl.BlockSpec((NCHUNK, BR, L), lambda i: (0, i, 0))
    return pl.pallas_call(
        _final_body,
        grid=(n // BR,),
        in_specs=[
            pl.BlockSpec((NCHUNK, BR, 2 * L), lambda i: (0, i, 0)),
            cblock, cblock, full((1, 64)),
        ],
        out_specs=pl.BlockSpec((BR, 64), lambda i: (i, 0)),
        out_shape=jax.ShapeDtypeStruct((n, 64), jnp.float32),
    )(acc, q2c, p2c, b2)


def _sc_body(u_hbm, q_hbm, src_hbm, dst_hbm, acc_hbm,
             accum, srcv, dstv, srcadj, dstadj, usv, udv, qsv, msgv, zerov,
             sem1, sem2, sem3):
    c = lax.axis_index("c")
    s = lax.axis_index("s")
    zeros = jnp.zeros((L,), jnp.float32)

    def zfill(r, _):
        zerov[r, 0:L] = zeros
        zerov[r, L:2 * L] = zeros
        return 0
    lax.fori_loop(0, ZB, zfill, 0)

    for j in range(NCHUNK // NCORE):
        chunk = c * (NCHUNK // NCORE) + j
        base_off = chunk * NPAD

        for z in range(ROWS_PT // ZB):
            pltpu.sync_copy(zerov,
                            accum.at[pl.ds(s * ROWS_PT + z * ZB, ZB)])
        plsc.subcore_barrier()

        def block_body(i, _):
            e0 = s * EPT + i * EB
            pltpu.sync_copy(src_hbm.at[pl.ds(e0, EB)], srcv)
            pltpu.sync_copy(dst_hbm.at[pl.ds(e0, EB)], dstv)

            def adj_body(r, _):
                sl = pl.ds(r * L, L)
                srcadj[sl] = srcv[sl] + base_off
                dstadj[sl] = dstv[sl] + base_off
                return 0
            lax.fori_loop(0, EB // L, adj_body, 0)

            cp1 = pltpu.async_copy(u_hbm.at[srcadj], usv, sem1)
            cp2 = pltpu.async_copy(u_hbm.at[dstadj], udv, sem2)
            cp3 = pltpu.async_copy(q_hbm.at[srcadj], qsv, sem3)
            cp1.wait()
            cp2.wait()
            cp3.wait()

            def row_body(r, _):
                ex = jnp.exp(udv[r, :] - usv[r, :])
                msgv[r, 0:L] = ex
                msgv[r, L:2 * L] = ex * qsv[r, :]
                return 0
            lax.fori_loop(0, EB, row_body, 0)

            pltpu.sync_copy(msgv, accum.at[dstv], add=True)
            return 0
        lax.fori_loop(0, EPT // EB, block_body, 0)

        plsc.subcore_barrier()
        pltpu.sync_copy(
            accum.at[pl.ds(s * ROWS_PT, ROWS_PT)],
            acc_hbm.at[pl.ds(base_off + s * ROWS_PT, ROWS_PT)])
        plsc.subcore_barrier()


@functools.cache
def _make_sc_pass():
    return pl.kernel(
        _sc_body,
        out_type=jax.ShapeDtypeStruct((NCHUNK * NPAD, 2 * L), jnp.float32),
        mesh=plsc.VectorSubcoreMesh(core_axis_name="c", subcore_axis_name="s",
                                    num_cores=NCORE, num_subcores=NSUB),
        compiler_params=pltpu.CompilerParams(use_tc_tiling_on_sc=False),
        scratch_types=[
        pltpu.MemorySpace.VMEM_SHARED((NPAD, 2 * L), jnp.float32),
        pltpu.MemorySpace.VMEM((EB,), jnp.int32),
        pltpu.MemorySpace.VMEM((EB,), jnp.int32),
        pltpu.MemorySpace.VMEM((EB,), jnp.int32),
        pltpu.MemorySpace.VMEM((EB,), jnp.int32),
        pltpu.MemorySpace.VMEM((EB, L), jnp.float32),
        pltpu.MemorySpace.VMEM((EB, L), jnp.float32),
        pltpu.MemorySpace.VMEM((EB, L), jnp.float32),
        pltpu.MemorySpace.VMEM((EB, 2 * L), jnp.float32),
        pltpu.MemorySpace.VMEM((ZB, 2 * L), jnp.float32),
            pltpu.SemaphoreType.DMA,
            pltpu.SemaphoreType.DMA,
            pltpu.SemaphoreType.DMA,
        ],
    )


def kernel(pos, attr, edge_index, W1_lin, W1_src, W1_dst, W1_pos, b1_pos,
           W2_lin, W2_src, W2_dst, W2_pos, b2_pos):
    N = pos.shape[0]
    E = edge_index.shape[1]
    assert N + 1 <= NPAD and E <= NSUB * EPT

    x = jnp.concatenate([pos, attr], axis=1)
    xpad = jnp.pad(x, ((0, NPAD - N), (0, 0)))
    src = edge_index[0].astype(jnp.int32)
    dst0 = edge_index[1].astype(jnp.int32)
    dst = jnp.where(src == dst0, jnp.int32(N), dst0)
    ep = NSUB * EPT
    src_p = jnp.pad(src, (0, ep - E), constant_values=N)
    dst_p = jnp.pad(dst, (0, ep - E), constant_values=N)

    sc_pass = _make_sc_pass()
    u1c, q1c, p1c = _prep1(xpad, W1_lin, W1_src, W1_pos)
    acc1 = sc_pass(u1c.reshape(NCHUNK * NPAD, L),
                   q1c.reshape(NCHUNK * NPAD, L), src_p, dst_p)
    u2c, q2c, p2c = _combine2(acc1.reshape(NCHUNK, NPAD, 2 * L), q1c, p1c,
                              b1_pos.reshape(1, 64), xpad,
                              W2_lin, W2_src, W2_pos)
    acc2 = sc_pass(u2c.reshape(NCHUNK * NPAD, L),
                   q2c.reshape(NCHUNK * NPAD, L), src_p, dst_p)
    out = _final(acc2.reshape(NCHUNK, NPAD, 2 * L), q2c, p2c,
                 b2_pos.reshape(1, 64))
    return out[:N]


# speedup vs baseline: 11.0714x; 2.2518x over previous
import functools

import jax
import jax.numpy as jnp
from jax import lax
from jax.experimental import pallas as pl
from jax.experimental.pallas import tpu as pltpu
from jax.experimental.pallas import tpu_sc as plsc

L = 16
NCHUNK = 4
NCORE = 2
NSUB = 16
NPAD = 50048
EPT = 51200
EB = 128
ROWS_PT = NPAD // NSUB
BR = 128


def _prep1_body(x_ref, wl_ref, ws_ref, wp_ref, u_ref, q_ref, p_ref):
    x = x_ref[...]
    pos = x[:, 0:3]
    P = jnp.dot(pos, wp_ref[...], preferred_element_type=jnp.float32)
    U = jnp.dot(x, ws_ref[...], preferred_element_type=jnp.float32) + P
    Q = jnp.dot(x, wl_ref[...], preferred_element_type=jnp.float32) - P
    for k in range(NCHUNK):
        sl = slice(L * k, L * k + L)
        u_ref[k] = U[:, sl]
        q_ref[k] = Q[:, sl]
        p_ref[k] = P[:, sl]


def _prep1(xpad, W_lin, W_src, W_pos):
    n = xpad.shape[0]
    cshape = jax.ShapeDtypeStruct((NCHUNK, n, L), jnp.float32)
    full = lambda s: pl.BlockSpec(s, lambda i: (0,) * len(s))
    return pl.pallas_call(
        _prep1_body,
        grid=(n // BR,),
        in_specs=[
            pl.BlockSpec((BR, 6), lambda i: (i, 0)),
            full((6, 64)), full((6, 64)), full((3, 64)),
        ],
        out_specs=[pl.BlockSpec((NCHUNK, BR, L), lambda i: (0, i, 0))] * 3,
        out_shape=[cshape] * 3,
    )(xpad, W_lin, W_src, W_pos)


def _combine2_body(acc_ref, q1_ref, p1_ref, b1_ref, x_ref, wl_ref, ws_ref,
                   wp_ref, u_ref, q_ref, p_ref):
    parts = []
    b1 = b1_ref[...]
    for k in range(NCHUNK):
        den = acc_ref[k][:, 0:L] + 1.0
        num = acc_ref[k][:, L:2 * L] + q1_ref[k]
        parts.append(num / den + p1_ref[k] + b1[:, L * k:L * k + L])
    h = jnp.concatenate(parts, axis=1)
    pos = x_ref[...][:, 0:3]
    P = jnp.dot(pos, wp_ref[...], preferred_element_type=jnp.float32)
    U = jnp.dot(h, ws_ref[...], preferred_element_type=jnp.float32) + P
    Q = jnp.dot(h, wl_ref[...], preferred_element_type=jnp.float32) - P
    for k in range(NCHUNK):
        sl = slice(L * k, L * k + L)
        u_ref[k] = U[:, sl]
        q_ref[k] = Q[:, sl]
        p_ref[k] = P[:, sl]


def _combine2(acc, q1c, p1c, b1, xpad, W_lin, W_src, W_pos):
    n = xpad.shape[0]
    cshape = jax.ShapeDtypeStruct((NCHUNK, n, L), jnp.float32)
    full = lambda s: pl.BlockSpec(s, lambda i: (0,) * len(s))
    cblock = pl.BlockSpec((NCHUNK, BR, L), lambda i: (0, i, 0))
    return pl.pallas_call(
        _combine2_body,
        grid=(n // BR,),
        in_specs=[
            pl.BlockSpec((NCHUNK, BR, 2 * L), lambda i: (0, i, 0)),
            cblock, cblock, full((1, 64)),
            pl.BlockSpec((BR, 6), lambda i: (i, 0)),
            full((64, 64)), full((64, 64)), full((3, 64)),
        ],
        out_specs=[cblock] * 3,
        out_shape=[cshape] * 3,
    )(acc, q1c, p1c, b1, xpad, W_lin, W_src, W_pos)


def _final_body(acc_ref, q2_ref, p2_ref, b2_ref, o_ref):
    b2 = b2_ref[...]
    for k in range(NCHUNK):
        den = acc_ref[k][:, 0:L] + 1.0
        num = acc_ref[k][:, L:2 * L] + q2_ref[k]
        o_ref[:, L * k:L * k + L] = num / den + p2_ref[k] + b2[:, L * k:L * k + L]


def _final(acc, q2c, p2c, b2):
    n = acc.shape[1]
    full = lambda s: pl.BlockSpec(s, lambda i: (0,) * len(s))
    cblock = pl.BlockSpec((NCHUNK, BR, L), lambda i: (0, i, 0))
    return pl.pallas_call(
        _final_body,
        grid=(n // BR,),
        in_specs=[
            pl.BlockSpec((NCHUNK, BR, 2 * L), lambda i: (0, i, 0)),
            cblock, cblock, full((1, 64)),
        ],
        out_specs=pl.BlockSpec((BR, 64), lambda i: (i, 0)),
        out_shape=jax.ShapeDtypeStruct((n, 64), jnp.float32),
    )(acc, q2c, p2c, b2)


NSB = EPT // EB
RPT2 = EPT // EB


def _sc_body(u_hbm, q_hbm, src_hbm, dst_hbm, acc_hbm,
             accum, srcb, drb, sab, dab, usb, udb, qsb, msgv,
             semi, semg):
    c = lax.axis_index("c")
    s = lax.axis_index("s")
    zeros = jnp.zeros((L,), jnp.float32)

    def fire_idx(g, b):
        r0 = s * RPT2 + g
        pltpu.async_copy(src_hbm.at[pl.ds(r0, 1)], srcb[b], semi[b])
        pltpu.async_copy(dst_hbm.at[pl.ds(r0, 1)], drb[b], semi[b])

    def wait_idx(g, b):
        r0 = s * RPT2 + g
        pltpu.make_async_copy(src_hbm.at[pl.ds(r0, 1)], srcb[b],
                              semi[b]).wait()
        pltpu.make_async_copy(dst_hbm.at[pl.ds(r0, 1)], drb[b],
                              semi[b]).wait()

    def adjust(b, base_off):
        @plsc.parallel_loop(0, EB // L, unroll=2)
        def _(r):
            sl = pl.ds(r * L, L)
            sab[b][0, sl] = srcb[b][0, sl] + base_off
            dab[b][0, sl] = drb[b][0, sl] + base_off

    def fire_gathers(b):
        pltpu.async_copy(u_hbm.at[sab[b].at[0]], usb[b], semg[b])
        pltpu.async_copy(u_hbm.at[dab[b].at[0]], udb[b], semg[b])
        pltpu.async_copy(q_hbm.at[sab[b].at[0]], qsb[b], semg[b])

    def wait_gathers(b):
        pltpu.make_async_copy(u_hbm.at[sab[b].at[0]], usb[b], semg[b]).wait()
        pltpu.make_async_copy(u_hbm.at[dab[b].at[0]], udb[b], semg[b]).wait()
        pltpu.make_async_copy(q_hbm.at[sab[b].at[0]], qsb[b], semg[b]).wait()

    def compute_scatter(b):
        @plsc.parallel_loop(0, EB, unroll=2)
        def _(r):
            ex = jnp.exp(udb[b][r, :] - usb[b][r, :])
            msgv[r, 0:L] = ex
            msgv[r, L:2 * L] = ex * qsb[b][r, :]
        pltpu.sync_copy(msgv, accum.at[drb[b].at[0]], add=True)

    nz = ROWS_PT // EB
    rem = ROWS_PT - nz * EB

    for j in range(NCHUNK // NCORE):
        chunk = c * (NCHUNK // NCORE) + j
        base_off = chunk * NPAD

        @plsc.parallel_loop(0, EB, unroll=2)
        def _(r):
            msgv[r, 0:L] = zeros
            msgv[r, L:2 * L] = zeros
        for z in range(nz):
            pltpu.sync_copy(msgv,
                            accum.at[pl.ds(s * ROWS_PT + z * EB, EB)])
        pltpu.sync_copy(msgv.at[pl.ds(0, rem)],
                        accum.at[pl.ds(s * ROWS_PT + nz * EB, rem)])
        plsc.subcore_barrier()

        fire_idx(0, 0)
        wait_idx(0, 0)
        adjust(0, base_off)
        fire_gathers(0)
        fire_idx(1, 1)

        def outer(t, _):
            for b in range(2):
                g = 2 * t + b
                b2 = 1 - b

                @pl.when(g + 1 < NSB)
                def _():
                    wait_idx(g + 1, b2)
                    adjust(b2, base_off)
                    fire_gathers(b2)

                wait_gathers(b)
                compute_scatter(b)

                @pl.when(g + 2 < NSB)
                def _():
                    fire_idx(g + 2, b)
            return 0
        lax.fori_loop(0, NSB // 2, outer, 0)

        plsc.subcore_barrier()
        pltpu.sync_copy(
            accum.at[pl.ds(s * ROWS_PT, ROWS_PT)],
            acc_hbm.at[pl.ds(base_off + s * ROWS_PT, ROWS_PT)])
        plsc.subcore_barrier()


@functools.cache
def _make_sc_pass():
    return pl.kernel(
        _sc_body,
        out_type=jax.ShapeDtypeStruct((NCHUNK * NPAD, 2 * L), jnp.float32),
        mesh=plsc.VectorSubcoreMesh(core_axis_name="c", subcore_axis_name="s",
                                    num_cores=NCORE, num_subcores=NSUB),
        compiler_params=pltpu.CompilerParams(use_tc_tiling_on_sc=False),
        scratch_types=[
            pltpu.MemorySpace.VMEM_SHARED((NPAD, 2 * L), jnp.float32),
            (pltpu.MemorySpace.VMEM((1, EB), jnp.int32),) * 2,
            (pltpu.MemorySpace.VMEM((1, EB), jnp.int32),) * 2,
            (pltpu.MemorySpace.VMEM((1, EB), jnp.int32),) * 2,
            (pltpu.MemorySpace.VMEM((1, EB), jnp.int32),) * 2,
            (pltpu.MemorySpace.VMEM((EB, L), jnp.float32),) * 2,
            (pltpu.MemorySpace.VMEM((EB, L), jnp.float32),) * 2,
            (pltpu.MemorySpace.VMEM((EB, L), jnp.float32),) * 2,
            pltpu.MemorySpace.VMEM((EB, 2 * L), jnp.float32),
            (pltpu.SemaphoreType.DMA,) * 2,
            (pltpu.SemaphoreType.DMA,) * 2,
        ],
    )


def kernel(pos, attr, edge_index, W1_lin, W1_src, W1_dst, W1_pos, b1_pos,
           W2_lin, W2_src, W2_dst, W2_pos, b2_pos):
    N = pos.shape[0]
    E = edge_index.shape[1]
    assert N + 1 <= NPAD and E <= NSUB * EPT

    x = jnp.concatenate([pos, attr], axis=1)
    xpad = jnp.pad(x, ((0, NPAD - N), (0, 0)))
    src = edge_index[0].astype(jnp.int32)
    dst0 = edge_index[1].astype(jnp.int32)
    dst = jnp.where(src == dst0, jnp.int32(N), dst0)
    ep = NSUB * EPT
    src_p = jnp.pad(src, (0, ep - E), constant_values=N).reshape(ep // EB, EB)
    dst_p = jnp.pad(dst, (0, ep - E), constant_values=N).reshape(ep // EB, EB)

    sc_pass = _make_sc_pass()
    u1c, q1c, p1c = _prep1(xpad, W1_lin, W1_src, W1_pos)
    acc1 = sc_pass(u1c.reshape(NCHUNK * NPAD, L),
                   q1c.reshape(NCHUNK * NPAD, L), src_p, dst_p)
    u2c, q2c, p2c = _combine2(acc1.reshape(NCHUNK, NPAD, 2 * L), q1c, p1c,
                              b1_pos.reshape(1, 64), xpad,
                              W2_lin, W2_src, W2_pos)
    acc2 = sc_pass(u2c.reshape(NCHUNK * NPAD, L),
                   q2c.reshape(NCHUNK * NPAD, L), src_p, dst_p)
    out = _final(acc2.reshape(NCHUNK, NPAD, 2 * L), q2c, p2c,
                 b2_pos.reshape(1, 64))
    return out[:N]


# speedup vs baseline: 13.7041x; 1.2378x over previous
import functools

import jax
import jax.numpy as jnp
from jax import lax
from jax.experimental import pallas as pl
from jax.experimental.pallas import tpu as pltpu
from jax.experimental.pallas import tpu_sc as plsc

L = 16
NCHUNK = 4
NCORE = 2
NSUB = 16
NPAD = 50176
EPT = 51200
EB = 128
ROWS_PT = NPAD // NSUB
BR = 1024


def _prep1_body(x_ref, wl_ref, ws_ref, wp_ref, ts_ref, u_ref, p_ref):
    x = x_ref[...]
    pos = x[:, 0:3]
    P = jnp.dot(pos, wp_ref[...], preferred_element_type=jnp.float32)
    U = jnp.dot(x, ws_ref[...], preferred_element_type=jnp.float32) + P
    Q = jnp.dot(x, wl_ref[...], preferred_element_type=jnp.float32) - P
    for k in range(NCHUNK):
        sl = slice(L * k, L * k + L)
        u_ref[k] = U[:, sl]
        p_ref[k] = P[:, sl]
        ts_ref[k] = jnp.concatenate([U[:, sl], Q[:, sl]], axis=1)


def _prep1(xpad, W_lin, W_src, W_pos):
    n = xpad.shape[0]
    full = lambda s: pl.BlockSpec(s, lambda i: (0,) * len(s))
    cblock = pl.BlockSpec((NCHUNK, BR, L), lambda i: (0, i, 0))
    tblock = pl.BlockSpec((NCHUNK, BR, 2 * L), lambda i: (0, i, 0))
    return pl.pallas_call(
        _prep1_body,
        grid=(n // BR,),
        in_specs=[
            pl.BlockSpec((BR, 6), lambda i: (i, 0)),
            full((6, 64)), full((6, 64)), full((3, 64)),
        ],
        out_specs=[tblock, cblock, cblock],
        out_shape=[jax.ShapeDtypeStruct((NCHUNK, n, 2 * L), jnp.float32),
                   jax.ShapeDtypeStruct((NCHUNK, n, L), jnp.float32),
                   jax.ShapeDtypeStruct((NCHUNK, n, L), jnp.float32)],
    )(xpad, W_lin, W_src, W_pos)


def _combine2_body(acc_ref, ts1_ref, p1_ref, b1_ref, x_ref, wl_ref, ws_ref,
                   wp_ref, ts_ref, u_ref, p_ref):
    parts = []
    b1 = b1_ref[...]
    for k in range(NCHUNK):
        den = acc_ref[k][:, 0:L] + 1.0
        num = acc_ref[k][:, L:2 * L] + ts1_ref[k][:, L:2 * L]
        parts.append(num / den + p1_ref[k] + b1[:, L * k:L * k + L])
    h = jnp.concatenate(parts, axis=1)
    pos = x_ref[...][:, 0:3]
    P = jnp.dot(pos, wp_ref[...], preferred_element_type=jnp.float32)
    U = jnp.dot(h, ws_ref[...], preferred_element_type=jnp.float32) + P
    Q = jnp.dot(h, wl_ref[...], preferred_element_type=jnp.float32) - P
    for k in range(NCHUNK):
        sl = slice(L * k, L * k + L)
        u_ref[k] = U[:, sl]
        p_ref[k] = P[:, sl]
        ts_ref[k] = jnp.concatenate([U[:, sl], Q[:, sl]], axis=1)


def _combine2(acc, ts1, p1c, b1, xpad, W_lin, W_src, W_pos):
    n = xpad.shape[0]
    full = lambda s: pl.BlockSpec(s, lambda i: (0,) * len(s))
    cblock = pl.BlockSpec((NCHUNK, BR, L), lambda i: (0, i, 0))
    tblock = pl.BlockSpec((NCHUNK, BR, 2 * L), lambda i: (0, i, 0))
    return pl.pallas_call(
        _combine2_body,
        grid=(n // BR,),
        in_specs=[
            tblock, tblock, cblock, full((1, 64)),
            pl.BlockSpec((BR, 6), lambda i: (i, 0)),
            full((64, 64)), full((64, 64)), full((3, 64)),
        ],
        out_specs=[tblock, cblock, cblock],
        out_shape=[jax.ShapeDtypeStruct((NCHUNK, n, 2 * L), jnp.float32),
                   jax.ShapeDtypeStruct((NCHUNK, n, L), jnp.float32),
                   jax.ShapeDtypeStruct((NCHUNK, n, L), jnp.float32)],
    )(acc, ts1, p1c, b1, xpad, W_lin, W_src, W_pos)


def _final_body(acc_ref, ts2_ref, p2_ref, b2_ref, o_ref):
    b2 = b2_ref[...]
    for k in range(NCHUNK):
        den = acc_ref[k][:, 0:L] + 1.0
        num = acc_ref[k][:, L:2 * L] + ts2_ref[k][:, L:2 * L]
        o_ref[:, L * k:L * k + L] = num / den + p2_ref[k] + b2[:, L * k:L * k + L]


def _final(acc, ts2, p2c, b2):
    n = acc.shape[1]
    full = lambda s: pl.BlockSpec(s, lambda i: (0,) * len(s))
    cblock = pl.BlockSpec((NCHUNK, BR, L), lambda i: (0, i, 0))
    tblock = pl.BlockSpec((NCHUNK, BR, 2 * L), lambda i: (0, i, 0))
    return pl.pallas_call(
        _final_body,
        grid=(n // BR,),
        in_specs=[tblock, tblock, cblock, full((1, 64))],
        out_specs=pl.BlockSpec((BR, 64), lambda i: (i, 0)),
        out_shape=jax.ShapeDtypeStruct((n, 64), jnp.float32),
    )(acc, ts2, p2c, b2)


NSB = EPT // EB
RPT2 = EPT // EB


def _sc_body(ts_hbm, u_hbm, idx_hbm, acc_hbm,
             accum, idxb, sab, dab, tsb, udb, msgv,
             semi, semg):
    c = lax.axis_index("c")
    s = lax.axis_index("s")
    zeros = jnp.zeros((L,), jnp.float32)

    def fire_idx(g, b):
        r0 = s * RPT2 + g
        pltpu.async_copy(idx_hbm.at[r0], idxb[b], semi[b])

    def wait_idx(g, b):
        r0 = s * RPT2 + g
        pltpu.make_async_copy(idx_hbm.at[r0], idxb[b], semi[b]).wait()

    def adjust(b, base_off):
        @plsc.parallel_loop(0, EB // L, unroll=2)
        def _(r):
            sl = pl.ds(r * L, L)
            sab[b][0, sl] = idxb[b][0, sl] + base_off
            dab[b][0, sl] = idxb[b][1, sl] + base_off

    def fire_gathers(b):
        pltpu.async_copy(ts_hbm.at[sab[b].at[0]], tsb[b], semg[b])
        pltpu.async_copy(u_hbm.at[dab[b].at[0]], udb[b], semg[b])

    def wait_gathers(b):
        pltpu.make_async_copy(ts_hbm.at[sab[b].at[0]], tsb[b], semg[b]).wait()
        pltpu.make_async_copy(u_hbm.at[dab[b].at[0]], udb[b], semg[b]).wait()

    def compute_scatter(b):
        @plsc.parallel_loop(0, EB, unroll=2)
        def _(r):
            ex = jnp.exp(udb[b][r, :] - tsb[b][r, 0:L])
            msgv[r, 0:L] = ex
            msgv[r, L:2 * L] = ex * tsb[b][r, L:2 * L]
        pltpu.sync_copy(msgv, accum.at[idxb[b].at[1]], add=True)

    nz = ROWS_PT // EB
    rem = ROWS_PT - nz * EB

    for j in range(NCHUNK // NCORE):
        chunk = c * (NCHUNK // NCORE) + j
        base_off = chunk * NPAD

        @plsc.parallel_loop(0, EB, unroll=2)
        def _(r):
            msgv[r, 0:L] = zeros
            msgv[r, L:2 * L] = zeros
        for z in range(nz):
            pltpu.sync_copy(msgv,
                            accum.at[pl.ds(s * ROWS_PT + z * EB, EB)])
        pltpu.sync_copy(msgv.at[pl.ds(0, rem)],
                        accum.at[pl.ds(s * ROWS_PT + nz * EB, rem)])
        plsc.subcore_barrier()

        fire_idx(0, 0)
        wait_idx(0, 0)
        adjust(0, base_off)
        fire_gathers(0)
        fire_idx(1, 1)

        def outer(t, _):
            for b in range(2):
                g = 2 * t + b
                b2 = 1 - b

                @pl.when(g + 1 < NSB)
                def _():
                    wait_idx(g + 1, b2)
                    adjust(b2, base_off)
                    fire_gathers(b2)

                wait_gathers(b)
                compute_scatter(b)

                @pl.when(g + 2 < NSB)
                def _():
                    fire_idx(g + 2, b)
            return 0
        lax.fori_loop(0, NSB // 2, outer, 0)

        plsc.subcore_barrier()
        pltpu.sync_copy(
            accum.at[pl.ds(s * ROWS_PT, ROWS_PT)],
            acc_hbm.at[pl.ds(base_off + s * ROWS_PT, ROWS_PT)])
        plsc.subcore_barrier()


@functools.cache
def _make_sc_pass():
    return pl.kernel(
        _sc_body,
        out_type=jax.ShapeDtypeStruct((NCHUNK * NPAD, 2 * L), jnp.float32),
        mesh=plsc.VectorSubcoreMesh(core_axis_name="c", subcore_axis_name="s",
                                    num_cores=NCORE, num_subcores=NSUB),
        compiler_params=pltpu.CompilerParams(use_tc_tiling_on_sc=False),
        scratch_types=[
            pltpu.MemorySpace.VMEM_SHARED((NPAD, 2 * L), jnp.float32),
            (pltpu.MemorySpace.VMEM((2, EB), jnp.int32),) * 2,
            (pltpu.MemorySpace.VMEM((1, EB), jnp.int32),) * 2,
            (pltpu.MemorySpace.VMEM((1, EB), jnp.int32),) * 2,
            (pltpu.MemorySpace.VMEM((EB, 2 * L), jnp.float32),) * 2,
            (pltpu.MemorySpace.VMEM((EB, L), jnp.float32),) * 2,
            pltpu.MemorySpace.VMEM((EB, 2 * L), jnp.float32),
            (pltpu.SemaphoreType.DMA,) * 2,
            (pltpu.SemaphoreType.DMA,) * 2,
        ],
    )


def kernel(pos, attr, edge_index, W1_lin, W1_src, W1_dst, W1_pos, b1_pos,
           W2_lin, W2_src, W2_dst, W2_pos, b2_pos):
    N = pos.shape[0]
    E = edge_index.shape[1]
    assert N + 1 <= NPAD and E <= NSUB * EPT

    x = jnp.concatenate([pos, attr], axis=1)
    xpad = jnp.pad(x, ((0, NPAD - N), (0, 0)))
    src = edge_index[0].astype(jnp.int32)
    dst0 = edge_index[1].astype(jnp.int32)
    dst = jnp.where(src == dst0, jnp.int32(N), dst0)
    ep = NSUB * EPT
    src_p = jnp.pad(src, (0, ep - E), constant_values=N).reshape(ep // EB, EB)
    dst_p = jnp.pad(dst, (0, ep - E), constant_values=N).reshape(ep // EB, EB)
    idx_p = jnp.stack([src_p, dst_p], axis=1)

    sc_pass = _make_sc_pass()
    ts1, u1c, p1c = _prep1(xpad, W1_lin, W1_src, W1_pos)
    acc1 = sc_pass(ts1.reshape(NCHUNK * NPAD, 2 * L),
                   u1c.reshape(NCHUNK * NPAD, L), idx_p)
    ts2, u2c, p2c = _combine2(acc1.reshape(NCHUNK, NPAD, 2 * L), ts1, p1c,
                              b1_pos.reshape(1, 64), xpad,
                              W2_lin, W2_src, W2_pos)
    acc2 = sc_pass(ts2.reshape(NCHUNK * NPAD, 2 * L),
                   u2c.reshape(NCHUNK * NPAD, L), idx_p)
    out = _final(acc2.reshape(NCHUNK, NPAD, 2 * L), ts2, p2c,
                 b2_pos.reshape(1, 64))
    return out[:N]


# speedup vs baseline: 16.0621x; 1.1721x over previous
import functools

import jax
import jax.numpy as jnp
from jax import lax
from jax.experimental import pallas as pl
from jax.experimental.pallas import tpu as pltpu
from jax.experimental.pallas import tpu_sc as plsc

L = 16
NCHUNK = 4
NCORE = 2
NSUB = 16
NPAD = 50176
EPT = 51200
EB = 128
ROWS_PT = NPAD // NSUB
BR = 1024


def _prep1_body(x_ref, wl_ref, ws_ref, wp_ref, ts_ref, p_ref):
    x = x_ref[...]
    pos = x[:, 0:3]
    P = jnp.dot(pos, wp_ref[...], preferred_element_type=jnp.float32)
    U = jnp.dot(x, ws_ref[...], preferred_element_type=jnp.float32) + P
    Q = jnp.dot(x, wl_ref[...], preferred_element_type=jnp.float32) - P
    W = jnp.exp(-U)
    for k in range(NCHUNK):
        sl = slice(L * k, L * k + L)
        p_ref[k] = P[:, sl]
        ts_ref[k] = jnp.concatenate([W[:, sl], W[:, sl] * Q[:, sl]], axis=1)


def _prep1(xpad, W_lin, W_src, W_pos):
    n = xpad.shape[0]
    full = lambda s: pl.BlockSpec(s, lambda i: (0,) * len(s))
    cblock = pl.BlockSpec((NCHUNK, BR, L), lambda i: (0, i, 0))
    tblock = pl.BlockSpec((NCHUNK, BR, 2 * L), lambda i: (0, i, 0))
    return pl.pallas_call(
        _prep1_body,
        grid=(n // BR,),
        in_specs=[
            pl.BlockSpec((BR, 6), lambda i: (i, 0)),
            full((6, 64)), full((6, 64)), full((3, 64)),
        ],
        out_specs=[tblock, cblock],
        out_shape=[jax.ShapeDtypeStruct((NCHUNK, n, 2 * L), jnp.float32),
                   jax.ShapeDtypeStruct((NCHUNK, n, L), jnp.float32)],
    )(xpad, W_lin, W_src, W_pos)


def _combine2_body(acc_ref, ts1_ref, p1_ref, b1_ref, x_ref, wl_ref, ws_ref,
                   wp_ref, ts_ref, p_ref):
    parts = []
    b1 = b1_ref[...]
    for k in range(NCHUNK):
        den = acc_ref[k][:, 0:L] + ts1_ref[k][:, 0:L]
        num = acc_ref[k][:, L:2 * L] + ts1_ref[k][:, L:2 * L]
        parts.append(num / den + p1_ref[k] + b1[:, L * k:L * k + L])
    h = jnp.concatenate(parts, axis=1)
    pos = x_ref[...][:, 0:3]
    P = jnp.dot(pos, wp_ref[...], preferred_element_type=jnp.float32)
    U = jnp.dot(h, ws_ref[...], preferred_element_type=jnp.float32) + P
    Q = jnp.dot(h, wl_ref[...], preferred_element_type=jnp.float32) - P
    W = jnp.exp(-U)
    for k in range(NCHUNK):
        sl = slice(L * k, L * k + L)
        p_ref[k] = P[:, sl]
        ts_ref[k] = jnp.concatenate([W[:, sl], W[:, sl] * Q[:, sl]], axis=1)


def _combine2(acc, ts1, p1c, b1, xpad, W_lin, W_src, W_pos):
    n = xpad.shape[0]
    full = lambda s: pl.BlockSpec(s, lambda i: (0,) * len(s))
    cblock = pl.BlockSpec((NCHUNK, BR, L), lambda i: (0, i, 0))
    tblock = pl.BlockSpec((NCHUNK, BR, 2 * L), lambda i: (0, i, 0))
    return pl.pallas_call(
        _combine2_body,
        grid=(n // BR,),
        in_specs=[
            tblock, tblock, cblock, full((1, 64)),
            pl.BlockSpec((BR, 6), lambda i: (i, 0)),
            full((64, 64)), full((64, 64)), full((3, 64)),
        ],
        out_specs=[tblock, cblock],
        out_shape=[jax.ShapeDtypeStruct((NCHUNK, n, 2 * L), jnp.float32),
                   jax.ShapeDtypeStruct((NCHUNK, n, L), jnp.float32)],
    )(acc, ts1, p1c, b1, xpad, W_lin, W_src, W_pos)


def _final_body(acc_ref, ts2_ref, p2_ref, b2_ref, o_ref):
    b2 = b2_ref[...]
    for k in range(NCHUNK):
        den = acc_ref[k][:, 0:L] + ts2_ref[k][:, 0:L]
        num = acc_ref[k][:, L:2 * L] + ts2_ref[k][:, L:2 * L]
        o_ref[:, L * k:L * k + L] = num / den + p2_ref[k] + b2[:, L * k:L * k + L]


def _final(acc, ts2, p2c, b2):
    n = acc.shape[1]
    full = lambda s: pl.BlockSpec(s, lambda i: (0,) * len(s))
    cblock = pl.BlockSpec((NCHUNK, BR, L), lambda i: (0, i, 0))
    tblock = pl.BlockSpec((NCHUNK, BR, 2 * L), lambda i: (0, i, 0))
    return pl.pallas_call(
        _final_body,
        grid=(n // BR,),
        in_specs=[tblock, tblock, cblock, full((1, 64))],
        out_specs=pl.BlockSpec((BR, 64), lambda i: (i, 0)),
        out_shape=jax.ShapeDtypeStruct((n, 64), jnp.float32),
    )(acc, ts2, p2c, b2)


NSB = EPT // EB
RPT2 = EPT // EB


def _sc_body(ts_hbm, idx_hbm, acc_hbm,
             accum, idxb, sab, tsb,
             semi, semg):
    c = lax.axis_index("c")
    s = lax.axis_index("s")
    zeros = jnp.zeros((L,), jnp.float32)

    def fire_idx(g, b):
        r0 = s * RPT2 + g
        pltpu.async_copy(idx_hbm.at[r0], idxb[b], semi[b])

    def wait_idx(g, b):
        r0 = s * RPT2 + g
        pltpu.make_async_copy(idx_hbm.at[r0], idxb[b], semi[b]).wait()

    def adjust(b, base_off):
        @plsc.parallel_loop(0, EB // L, unroll=2)
        def _(r):
            sl = pl.ds(r * L, L)
            sab[b][0, sl] = idxb[b][0, sl] + base_off

    def fire_gathers(b):
        pltpu.async_copy(ts_hbm.at[sab[b].at[0]], tsb[b], semg[b])

    def wait_gathers(b):
        pltpu.make_async_copy(ts_hbm.at[sab[b].at[0]], tsb[b], semg[b]).wait()

    def compute_scatter(b):
        pltpu.sync_copy(tsb[b], accum.at[idxb[b].at[1]], add=True)

    nz = ROWS_PT // EB
    rem = ROWS_PT - nz * EB

    for j in range(NCHUNK // NCORE):
        chunk = c * (NCHUNK // NCORE) + j
        base_off = chunk * NPAD

        @plsc.parallel_loop(0, EB, unroll=2)
        def _(r):
            tsb[0][r, 0:L] = zeros
            tsb[0][r, L:2 * L] = zeros
        for z in range(nz):
            pltpu.sync_copy(tsb[0],
                            accum.at[pl.ds(s * ROWS_PT + z * EB, EB)])
        pltpu.sync_copy(tsb[0].at[pl.ds(0, rem)],
                        accum.at[pl.ds(s * ROWS_PT + nz * EB, rem)])
        plsc.subcore_barrier()

        fire_idx(0, 0)
        wait_idx(0, 0)
        adjust(0, base_off)
        fire_gathers(0)
        fire_idx(1, 1)

        def outer(t, _):
            for b in range(2):
                g = 2 * t + b
                b2 = 1 - b

                @pl.when(g + 1 < NSB)
                def _():
                    wait_idx(g + 1, b2)
                    adjust(b2, base_off)
                    fire_gathers(b2)

                wait_gathers(b)
                compute_scatter(b)

                @pl.when(g + 2 < NSB)
                def _():
                    fire_idx(g + 2, b)
            return 0
        lax.fori_loop(0, NSB // 2, outer, 0)

        plsc.subcore_barrier()
        pltpu.sync_copy(
            accum.at[pl.ds(s * ROWS_PT, ROWS_PT)],
            acc_hbm.at[pl.ds(base_off + s * ROWS_PT, ROWS_PT)])
        plsc.subcore_barrier()


@functools.cache
def _make_sc_pass():
    return pl.kernel(
        _sc_body,
        out_type=jax.ShapeDtypeStruct((NCHUNK * NPAD, 2 * L), jnp.float32),
        mesh=plsc.VectorSubcoreMesh(core_axis_name="c", subcore_axis_name="s",
                                    num_cores=NCORE, num_subcores=NSUB),
        compiler_params=pltpu.CompilerParams(use_tc_tiling_on_sc=False),
        scratch_types=[
            pltpu.MemorySpace.VMEM_SHARED((NPAD, 2 * L), jnp.float32),
            (pltpu.MemorySpace.VMEM((2, EB), jnp.int32),) * 2,
            (pltpu.MemorySpace.VMEM((1, EB), jnp.int32),) * 2,
            (pltpu.MemorySpace.VMEM((EB, 2 * L), jnp.float32),) * 2,
            (pltpu.SemaphoreType.DMA,) * 2,
            (pltpu.SemaphoreType.DMA,) * 2,
        ],
    )


def kernel(pos, attr, edge_index, W1_lin, W1_src, W1_dst, W1_pos, b1_pos,
           W2_lin, W2_src, W2_dst, W2_pos, b2_pos):
    N = pos.shape[0]
    E = edge_index.shape[1]
    assert N + 1 <= NPAD and E <= NSUB * EPT

    x = jnp.concatenate([pos, attr], axis=1)
    xpad = jnp.pad(x, ((0, NPAD - N), (0, 0)))
    src = edge_index[0].astype(jnp.int32)
    dst0 = edge_index[1].astype(jnp.int32)
    dst = jnp.where(src == dst0, jnp.int32(N), dst0)
    ep = NSUB * EPT
    src_p = jnp.pad(src, (0, ep - E), constant_values=N).reshape(ep // EB, EB)
    dst_p = jnp.pad(dst, (0, ep - E), constant_values=N).reshape(ep // EB, EB)
    idx_p = jnp.stack([src_p, dst_p], axis=1)

    sc_pass = _make_sc_pass()
    ts1, p1c = _prep1(xpad, W1_lin, W1_src, W1_pos)
    acc1 = sc_pass(ts1.reshape(NCHUNK * NPAD, 2 * L), idx_p)
    ts2, p2c = _combine2(acc1.reshape(NCHUNK, NPAD, 2 * L), ts1, p1c,
                         b1_pos.reshape(1, 64), xpad,
                         W2_lin, W2_src, W2_pos)
    acc2 = sc_pass(ts2.reshape(NCHUNK * NPAD, 2 * L), idx_p)
    out = _final(acc2.reshape(NCHUNK, NPAD, 2 * L), ts2, p2c,
                 b2_pos.reshape(1, 64))
    return out[:N]


# speedup vs baseline: 17.5180x; 1.0906x over previous
import functools

import jax
import jax.numpy as jnp
from jax import lax
from jax.experimental import pallas as pl
from jax.experimental.pallas import tpu as pltpu
from jax.experimental.pallas import tpu_sc as plsc

L = 16
NCHUNK = 4
NCORE = 2
NSUB = 16
NPAD = 50176
EPT = 51200
EB = 128
ROWS_PT = NPAD // NSUB
BR = 1024


def _prep1_body(x_ref, wl_ref, ws_ref, wp_ref, ts_ref, p_ref):
    x = x_ref[...]
    pos = x[:, 0:3]
    P = jnp.dot(pos, wp_ref[...], preferred_element_type=jnp.float32)
    U = jnp.dot(x, ws_ref[...], preferred_element_type=jnp.float32) + P
    Q = jnp.dot(x, wl_ref[...], preferred_element_type=jnp.float32) - P
    W = jnp.exp(-U)
    for k in range(NCHUNK):
        sl = slice(L * k, L * k + L)
        p_ref[k] = P[:, sl]
        ts_ref[k] = jnp.concatenate([W[:, sl], W[:, sl] * Q[:, sl]], axis=1)


def _prep1(xpad, W_lin, W_src, W_pos):
    n = xpad.shape[0]
    full = lambda s: pl.BlockSpec(s, lambda i: (0,) * len(s))
    cblock = pl.BlockSpec((NCHUNK, BR, L), lambda i: (0, i, 0))
    tblock = pl.BlockSpec((NCHUNK, BR, 2 * L), lambda i: (0, i, 0))
    return pl.pallas_call(
        _prep1_body,
        grid=(n // BR,),
        in_specs=[
            pl.BlockSpec((BR, 6), lambda i: (i, 0)),
            full((6, 64)), full((6, 64)), full((3, 64)),
        ],
        out_specs=[tblock, cblock],
        out_shape=[jax.ShapeDtypeStruct((NCHUNK, n, 2 * L), jnp.float32),
                   jax.ShapeDtypeStruct((NCHUNK, n, L), jnp.float32)],
    )(xpad, W_lin, W_src, W_pos)


def _combine2_body(acc_ref, ts1_ref, p1_ref, b1_ref, x_ref, wl_ref, ws_ref,
                   wp_ref, ts_ref, p_ref):
    parts = []
    b1 = b1_ref[...]
    for k in range(NCHUNK):
        den = acc_ref[k][:, 0:L] + ts1_ref[k][:, 0:L]
        num = acc_ref[k][:, L:2 * L] + ts1_ref[k][:, L:2 * L]
        parts.append(num / den + p1_ref[k] + b1[:, L * k:L * k + L])
    h = jnp.concatenate(parts, axis=1)
    pos = x_ref[...][:, 0:3]
    P = jnp.dot(pos, wp_ref[...], preferred_element_type=jnp.float32)
    U = jnp.dot(h, ws_ref[...], preferred_element_type=jnp.float32) + P
    Q = jnp.dot(h, wl_ref[...], preferred_element_type=jnp.float32) - P
    W = jnp.exp(-U)
    for k in range(NCHUNK):
        sl = slice(L * k, L * k + L)
        p_ref[k] = P[:, sl]
        ts_ref[k] = jnp.concatenate([W[:, sl], W[:, sl] * Q[:, sl]], axis=1)


def _combine2(acc, ts1, p1c, b1, xpad, W_lin, W_src, W_pos):
    n = xpad.shape[0]
    full = lambda s: pl.BlockSpec(s, lambda i: (0,) * len(s))
    cblock = pl.BlockSpec((NCHUNK, BR, L), lambda i: (0, i, 0))
    tblock = pl.BlockSpec((NCHUNK, BR, 2 * L), lambda i: (0, i, 0))
    return pl.pallas_call(
        _combine2_body,
        grid=(n // BR,),
        in_specs=[
            tblock, tblock, cblock, full((1, 64)),
            pl.BlockSpec((BR, 6), lambda i: (i, 0)),
            full((64, 64)), full((64, 64)), full((3, 64)),
        ],
        out_specs=[tblock, cblock],
        out_shape=[jax.ShapeDtypeStruct((NCHUNK, n, 2 * L), jnp.float32),
                   jax.ShapeDtypeStruct((NCHUNK, n, L), jnp.float32)],
    )(acc, ts1, p1c, b1, xpad, W_lin, W_src, W_pos)


def _final_body(acc_ref, ts2_ref, p2_ref, b2_ref, o_ref):
    b2 = b2_ref[...]
    for k in range(NCHUNK):
        den = acc_ref[k][:, 0:L] + ts2_ref[k][:, 0:L]
        num = acc_ref[k][:, L:2 * L] + ts2_ref[k][:, L:2 * L]
        o_ref[:, L * k:L * k + L] = num / den + p2_ref[k] + b2[:, L * k:L * k + L]


def _final(acc, ts2, p2c, b2):
    n = acc.shape[1]
    full = lambda s: pl.BlockSpec(s, lambda i: (0,) * len(s))
    cblock = pl.BlockSpec((NCHUNK, BR, L), lambda i: (0, i, 0))
    tblock = pl.BlockSpec((NCHUNK, BR, 2 * L), lambda i: (0, i, 0))
    return pl.pallas_call(
        _final_body,
        grid=(n // BR,),
        in_specs=[tblock, tblock, cblock, full((1, 64))],
        out_specs=pl.BlockSpec((BR, 64), lambda i: (i, 0)),
        out_shape=jax.ShapeDtypeStruct((n, 64), jnp.float32),
    )(acc, ts2, p2c, b2)


SUBB = 2
NSB = EPT // (EB * SUBB)
RPT2 = EPT // EB


def _sc_body(ts_hbm, idx_hbm, acc_hbm,
             accum, idxb, sab, tsb,
             semi, semg):
    c = lax.axis_index("c")
    s = lax.axis_index("s")
    zeros = jnp.zeros((L,), jnp.float32)

    def fire_idx(g, b):
        r0 = s * RPT2 + g * SUBB
        pltpu.async_copy(idx_hbm.at[pl.ds(r0, SUBB)], idxb[b], semi[b])

    def wait_idx(g, b):
        r0 = s * RPT2 + g * SUBB
        pltpu.make_async_copy(idx_hbm.at[pl.ds(r0, SUBB)], idxb[b],
                              semi[b]).wait()

    def adjust(b, base_off):
        for jj in range(SUBB):
            @plsc.parallel_loop(0, EB // L, unroll=2)
            def _(r):
                sl = pl.ds(r * L, L)
                sab[b][jj, sl] = idxb[b][jj, 0, sl] + base_off

    def fire_gathers(b):
        for jj in range(SUBB):
            pltpu.async_copy(ts_hbm.at[sab[b].at[jj]], tsb[b].at[jj],
                             semg[b])

    def wait_gathers(b):
        for jj in range(SUBB):
            pltpu.make_async_copy(ts_hbm.at[sab[b].at[jj]], tsb[b].at[jj],
                                  semg[b]).wait()

    def compute_scatter(b):
        for jj in range(SUBB):
            pltpu.sync_copy(tsb[b].at[jj], accum.at[idxb[b].at[jj, 1]],
                            add=True)

    nz = ROWS_PT // EB
    rem = ROWS_PT - nz * EB

    for j in range(NCHUNK // NCORE):
        chunk = c * (NCHUNK // NCORE) + j
        base_off = chunk * NPAD

        @plsc.parallel_loop(0, EB, unroll=2)
        def _(r):
            tsb[0][0, r, 0:L] = zeros
            tsb[0][0, r, L:2 * L] = zeros
        for z in range(nz):
            pltpu.sync_copy(tsb[0].at[0],
                            accum.at[pl.ds(s * ROWS_PT + z * EB, EB)])
        pltpu.sync_copy(tsb[0].at[0, pl.ds(0, rem)],
                        accum.at[pl.ds(s * ROWS_PT + nz * EB, rem)])
        plsc.subcore_barrier()

        fire_idx(0, 0)
        wait_idx(0, 0)
        adjust(0, base_off)
        fire_gathers(0)
        fire_idx(1, 1)

        def outer(t, _):
            for b in range(2):
                g = 2 * t + b
                b2 = 1 - b

                @pl.when(g + 1 < NSB)
                def _():
                    wait_idx(g + 1, b2)
                    adjust(b2, base_off)
                    fire_gathers(b2)

                wait_gathers(b)
                compute_scatter(b)

                @pl.when(g + 2 < NSB)
                def _():
                    fire_idx(g + 2, b)
            return 0
        lax.fori_loop(0, NSB // 2, outer, 0)

        plsc.subcore_barrier()
        pltpu.sync_copy(
            accum.at[pl.ds(s * ROWS_PT, ROWS_PT)],
            acc_hbm.at[pl.ds(base_off + s * ROWS_PT, ROWS_PT)])
        plsc.subcore_barrier()


@functools.cache
def _make_sc_pass():
    return pl.kernel(
        _sc_body,
        out_type=jax.ShapeDtypeStruct((NCHUNK * NPAD, 2 * L), jnp.float32),
        mesh=plsc.VectorSubcoreMesh(core_axis_name="c", subcore_axis_name="s",
                                    num_cores=NCORE, num_subcores=NSUB),
        compiler_params=pltpu.CompilerParams(use_tc_tiling_on_sc=False),
        scratch_types=[
            pltpu.MemorySpace.VMEM_SHARED((NPAD, 2 * L), jnp.float32),
            (pltpu.MemorySpace.VMEM((SUBB, 2, EB), jnp.int32),) * 2,
            (pltpu.MemorySpace.VMEM((SUBB, EB), jnp.int32),) * 2,
            (pltpu.MemorySpace.VMEM((SUBB, EB, 2 * L), jnp.float32),) * 2,
            (pltpu.SemaphoreType.DMA,) * 2,
            (pltpu.SemaphoreType.DMA,) * 2,
        ],
    )


def kernel(pos, attr, edge_index, W1_lin, W1_src, W1_dst, W1_pos, b1_pos,
           W2_lin, W2_src, W2_dst, W2_pos, b2_pos):
    N = pos.shape[0]
    E = edge_index.shape[1]
    assert N + 1 <= NPAD and E <= NSUB * EPT

    x = jnp.concatenate([pos, attr], axis=1)
    xpad = jnp.pad(x, ((0, NPAD - N), (0, 0)))
    src = edge_index[0].astype(jnp.int32)
    dst0 = edge_index[1].astype(jnp.int32)
    dst = jnp.where(src == dst0, jnp.int32(N), dst0)
    ep = NSUB * EPT
    src_p = jnp.pad(src, (0, ep - E), constant_values=N).reshape(ep // EB, EB)
    dst_p = jnp.pad(dst, (0, ep - E), constant_values=N).reshape(ep // EB, EB)
    idx_p = jnp.stack([src_p, dst_p], axis=1)

    sc_pass = _make_sc_pass()
    ts1, p1c = _prep1(xpad, W1_lin, W1_src, W1_pos)
    acc1 = sc_pass(ts1.reshape(NCHUNK * NPAD, 2 * L), idx_p)
    ts2, p2c = _combine2(acc1.reshape(NCHUNK, NPAD, 2 * L), ts1, p1c,
                         b1_pos.reshape(1, 64), xpad,
                         W2_lin, W2_src, W2_pos)
    acc2 = sc_pass(ts2.reshape(NCHUNK * NPAD, 2 * L), idx_p)
    out = _final(acc2.reshape(NCHUNK, NPAD, 2 * L), ts2, p2c,
                 b2_pos.reshape(1, 64))
    return out[:N]
